# baseline (speedup 1.0000x reference)
import functools

import jax
import jax.numpy as jnp
from jax import lax
from jax.experimental import pallas as pl
from jax.experimental.pallas import tpu as pltpu
from jax.experimental.pallas import tpu_sc as plsc

VOCAB = 1000000
N_EMBD = 64
SEQ_LEN = 2048
BATCH = 4

NC = 2
NS = 16
NW = NC * NS
B = BATCH * SEQ_LEN
BPW = B // NW
CHUNK = 128
NCHUNK = BPW // CHUNK
LANES = 16
SEGS = N_EMBD // LANES

_mesh = plsc.VectorSubcoreMesh(core_axis_name="c", subcore_axis_name="s")


@functools.partial(
    pl.kernel,
    out_type=jax.ShapeDtypeStruct((B, N_EMBD), jnp.float32),
    mesh=_mesh,
    scratch_types=[
        pltpu.VMEM((NCHUNK, CHUNK), jnp.int32),
        pltpu.VMEM((BPW, N_EMBD), jnp.float32),
        pltpu.VMEM((BPW, N_EMBD), jnp.float32),
        pltpu.SemaphoreType.DMA,
    ],
    compiler_params=pltpu.CompilerParams(use_tc_tiling_on_sc=False),
)
def _embed_sc(x_hbm, tok_hbm, pos_hbm, out_hbm, idx_v, rows_v, pos_v, sem):
    wid = lax.axis_index("s") * NC + lax.axis_index("c")
    base = pl.multiple_of(wid * BPW, BPW)

    pltpu.sync_copy(x_hbm.at[pl.ds(wid * NCHUNK, NCHUNK)], idx_v)
    copies = [
        pltpu.async_copy(
            tok_hbm.at[idx_v.at[c]],
            rows_v.at[pl.ds(c * CHUNK, CHUNK)],
            sem,
        )
        for c in range(NCHUNK)
    ]

    pos_base = pl.multiple_of(lax.rem(wid * BPW, SEQ_LEN), BPW)
    pltpu.sync_copy(pos_hbm.at[pl.ds(pos_base, BPW)], pos_v)

    for c in copies:
        c.wait()

    def add_row(i, _):
        for j in range(SEGS):
            sl = pl.ds(j * LANES, LANES)
            rows_v[i, sl] = rows_v[i, sl] + pos_v[i, sl]
        return 0

    lax.fori_loop(0, BPW, add_row, 0)

    pltpu.sync_copy(rows_v, out_hbm.at[pl.ds(base, BPW)])


def kernel(x, tok_table, pos_table):
    x2d = x.reshape(NW * NCHUNK, CHUNK).astype(jnp.int32)
    out = _embed_sc(x2d, tok_table, pos_table)
    return out.reshape(BATCH, SEQ_LEN, N_EMBD)


# speedup vs baseline: 3.4710x; 3.4710x over previous
import functools

import jax
import jax.numpy as jnp
from jax import lax
from jax.experimental import pallas as pl
from jax.experimental.pallas import tpu as pltpu
from jax.experimental.pallas import tpu_sc as plsc

VOCAB = 1000000
N_EMBD = 64
SEQ_LEN = 2048
BATCH = 4

NC = 2
NS = 16
NW = NC * NS
B = BATCH * SEQ_LEN
BPW = B // NW
LANES = 16
SEGS = N_EMBD // LANES
RING = 4
VGRP = BPW // LANES
TILE_W = 128
LAST_FULL = (VOCAB // TILE_W) * TILE_W - TILE_W
SAFE_MAX = LAST_FULL + TILE_W - 1
TAIL_BASE = LAST_FULL + TILE_W

_mesh = plsc.VectorSubcoreMesh(core_axis_name="c", subcore_axis_name="s")


@functools.partial(
    pl.kernel,
    out_type=jax.ShapeDtypeStruct((BATCH, N_EMBD, SEQ_LEN), jnp.float32),
    mesh=_mesh,
    scratch_types=[
        pltpu.VMEM((BPW,), jnp.int32),
        pltpu.VMEM((RING, N_EMBD, TILE_W), jnp.float32),
        pltpu.VMEM((N_EMBD, BPW), jnp.float32),
        pltpu.VMEM((N_EMBD, BPW), jnp.float32),
        pltpu.VMEM((N_EMBD, VOCAB - TAIL_BASE), jnp.float32),
        pltpu.SemaphoreType.DMA,
    ],
    compiler_params=pltpu.CompilerParams(
        use_tc_tiling_on_sc=True, needs_layout_passes=False),
)
def _embed_sc(xf_hbm, tokT_hbm, posT_hbm, out_hbm, idx_v, blocks_v, fbuf,
              pbuf, tail_v, sem):
    wid = lax.axis_index("s") * NC + lax.axis_index("c")
    base = pl.multiple_of(wid * BPW, BPW)
    b = wid // (NW // BATCH)
    t0 = pl.multiple_of(lax.rem(base, SEQ_LEN), BPW)

    pltpu.sync_copy(xf_hbm.at[pl.ds(base, BPW)], idx_v)
    pltpu.sync_copy(posT_hbm.at[:, pl.ds(t0, BPW)], pbuf)

    lane_iota = lax.iota(jnp.int32, LANES)

    def extract(src_ref, ring_slot, lane, k):
        lane_s = jnp.full((LANES,), lane, jnp.int32)
        k_s = jnp.full((LANES,), k, jnp.int32)
        for f in range(SEGS):
            feat = lane_iota + (f * LANES)
            if ring_slot is None:
                tv = plsc.load_gather(src_ref, [feat, lane_s])
            else:
                slot_s = jnp.full((LANES,), ring_slot, jnp.int32)
                tv = plsc.load_gather(src_ref, [slot_s, feat, lane_s])
            pv = plsc.load_gather(pbuf, [feat, k_s])
            plsc.store_scatter(fbuf, [feat, k_s], tv + pv)

    def group(g, _):
        idx_vec = idx_v[pl.ds(g * LANES, LANES)]
        safe_vec = jnp.minimum(idx_vec, SAFE_MAX)
        for q in range(LANES // RING):
            copies = []
            for j in range(RING):
                tok = safe_vec[q * RING + j]
                blk = pl.multiple_of((tok >> 7) * TILE_W, TILE_W)
                copies.append(pltpu.async_copy(
                    tokT_hbm.at[:, pl.ds(blk, TILE_W)],
                    blocks_v.at[j], sem))
            for c in copies:
                c.wait()
            for j in range(RING):
                tok = safe_vec[q * RING + j]
                extract(blocks_v, j, tok & (TILE_W - 1),
                        g * LANES + q * RING + j)
        return 0

    lax.fori_loop(0, VGRP, group, 0)

    def tail_group(g, _):
        idx_vec = idx_v[pl.ds(g * LANES, LANES)]
        any_tail = jnp.max(idx_vec) >= TAIL_BASE

        @pl.when(any_tail)
        def _():
            pltpu.sync_copy(tokT_hbm.at[:, pl.ds(TAIL_BASE, VOCAB - TAIL_BASE)],
                            tail_v)
            for j in range(LANES):
                tok = idx_vec[j]

                @pl.when(tok >= TAIL_BASE)
                def _():
                    extract(tail_v, None, tok - TAIL_BASE, g * LANES + j)

        return 0

    lax.fori_loop(0, VGRP, tail_group, 0)

    pltpu.sync_copy(fbuf, out_hbm.at[b, :, pl.ds(t0, BPW)])


def kernel(x, tok_table, pos_table):
    xf = x.reshape(B).astype(jnp.int32)
    out_fm = _embed_sc(xf, tok_table.T, pos_table.T)
    return out_fm.transpose(0, 2, 1)


# speedup vs baseline: 4.2079x; 1.2123x over previous
import functools

import jax
import jax.numpy as jnp
from jax import lax
from jax.experimental import pallas as pl
from jax.experimental.pallas import tpu as pltpu
from jax.experimental.pallas import tpu_sc as plsc

VOCAB = 1000000
N_EMBD = 64
SEQ_LEN = 2048
BATCH = 4

NC = 2
NS = 16
NW = NC * NS
B = BATCH * SEQ_LEN
BPW = B // NW
LANES = 16
SEGS = N_EMBD // LANES
QUAD = 4
NQUAD = 8
RING = 2 * QUAD
VGRP = BPW // LANES
PGRP = BPW // (QUAD * NQUAD)
TILE_W = 128
LAST_FULL = (VOCAB // TILE_W) * TILE_W - TILE_W
SAFE_MAX = LAST_FULL + TILE_W - 1
TAIL_BASE = LAST_FULL + TILE_W

_mesh = plsc.VectorSubcoreMesh(core_axis_name="c", subcore_axis_name="s")


@functools.partial(
    pl.kernel,
    out_type=jax.ShapeDtypeStruct((BATCH, N_EMBD, SEQ_LEN), jnp.float32),
    mesh=_mesh,
    scratch_types=[
        pltpu.VMEM((BPW,), jnp.int32),
        pltpu.VMEM((RING, N_EMBD, TILE_W), jnp.float32),
        pltpu.VMEM((N_EMBD, BPW), jnp.float32),
        pltpu.VMEM((N_EMBD, BPW), jnp.float32),
        pltpu.VMEM((N_EMBD, VOCAB - TAIL_BASE), jnp.float32),
        pltpu.SemaphoreType.DMA,
        pltpu.SemaphoreType.DMA,
    ],
    compiler_params=pltpu.CompilerParams(
        use_tc_tiling_on_sc=True, needs_layout_passes=False),
)
def _embed_sc(xf_hbm, tokT_hbm, posT_hbm, out_hbm, idx_v, blocks_v, fbuf,
              pbuf, tail_v, sem_a, sem_b):
    wid = lax.axis_index("s") * NC + lax.axis_index("c")
    base = pl.multiple_of(wid * BPW, BPW)
    b = wid // (NW // BATCH)
    t0 = pl.multiple_of(lax.rem(base, SEQ_LEN), BPW)

    pltpu.sync_copy(xf_hbm.at[pl.ds(base, BPW)], idx_v)
    pltpu.sync_copy(posT_hbm.at[:, pl.ds(t0, BPW)], pbuf)

    lane_iota = lax.iota(jnp.int32, LANES)

    def extract(src_ref, ring_slot, lane, k):
        lane_s = jnp.full((LANES,), lane, jnp.int32)
        k_s = jnp.full((LANES,), k, jnp.int32)
        for f in range(SEGS):
            feat = lane_iota + (f * LANES)
            if ring_slot is None:
                tv = plsc.load_gather(src_ref, [feat, lane_s])
            else:
                slot_s = jnp.full((LANES,), ring_slot, jnp.int32)
                tv = plsc.load_gather(src_ref, [slot_s, feat, lane_s])
            pv = plsc.load_gather(pbuf, [feat, k_s])
            plsc.store_scatter(fbuf, [feat, k_s], tv + pv)

    sems = (sem_a, sem_b)

    def group(g, _):
        vec_a = jnp.minimum(idx_v[pl.ds(g * 2 * LANES, LANES)], SAFE_MAX)
        vec_b = jnp.minimum(
            idx_v[pl.ds(g * 2 * LANES + LANES, LANES)], SAFE_MAX)

        def quad_tok(q, j):
            lane = q * QUAD + j
            vec = vec_a if lane < LANES else vec_b
            return vec[lane % LANES]

        def fire(q):
            half = (q % 2) * QUAD
            cps = []
            for j in range(QUAD):
                tok = quad_tok(q, j)
                blk = pl.multiple_of((tok >> 7) * TILE_W, TILE_W)
                cps.append(pltpu.async_copy(
                    tokT_hbm.at[:, pl.ds(blk, TILE_W)],
                    blocks_v.at[half + j], sems[q % 2]))
            return cps

        pending = fire(0)
        for q in range(NQUAD):
            nxt = fire(q + 1) if q + 1 < NQUAD else None
            for c in pending:
                c.wait()
            half = (q % 2) * QUAD
            for j in range(QUAD):
                tok = quad_tok(q, j)
                extract(blocks_v, half + j, tok & (TILE_W - 1),
                        g * QUAD * NQUAD + q * QUAD + j)
            pending = nxt
        return 0

    lax.fori_loop(0, PGRP, group, 0)

    def tail_group(g, _):
        idx_vec = idx_v[pl.ds(g * LANES, LANES)]
        any_tail = jnp.max(idx_vec) >= TAIL_BASE

        @pl.when(any_tail)
        def _():
            pltpu.sync_copy(tokT_hbm.at[:, pl.ds(TAIL_BASE, VOCAB - TAIL_BASE)],
                            tail_v)
            for j in range(LANES):
                tok = idx_vec[j]

                @pl.when(tok >= TAIL_BASE)
                def _():
                    extract(tail_v, None, tok - TAIL_BASE, g * LANES + j)

        return 0

    lax.fori_loop(0, VGRP, tail_group, 0)

    pltpu.sync_copy(fbuf, out_hbm.at[b, :, pl.ds(t0, BPW)])


def kernel(x, tok_table, pos_table):
    xf = x.reshape(B).astype(jnp.int32)
    out_fm = _embed_sc(xf, tok_table.T, pos_table.T)
    return out_fm.transpose(0, 2, 1)
